# baseline (speedup 1.0000x reference)
import functools

import jax
import jax.numpy as jnp
from jax import lax
from jax.experimental import pallas as pl
from jax.experimental.pallas import tpu as pltpu
from jax.experimental.pallas import tpu_sc as plsc

_N = 10000
_E = 320000
_NODE_DIM = 128
_HID = 64
_G = 64
_EPS = 1e-5

_NC = 2
_NS = 16
_NW = _NC * _NS
_CH = 128
_NCHUNK = 80
_EPT = _NCHUNK * _CH
_E_PAD = _NW * _EPT
_N_PAD = 10240
_RPS = _N_PAD // _NS
_NBUF = 8
_LOOK = 4
_HH = _HID // 2
_NCHUNK_A = 160
_EPS_A = _NCHUNK_A * _CH

_mesh = plsc.VectorSubcoreMesh(
    core_axis_name="c", subcore_axis_name="s", num_cores=_NC, num_subcores=_NS)
_sc_params = pltpu.CompilerParams(use_tc_tiling_on_sc=False)


_DW = 8


@functools.partial(
    pl.kernel,
    out_type=jax.ShapeDtypeStruct((_NC, _N_PAD, _DW), jnp.float32),
    mesh=_mesh,
    scratch_types=[
        pltpu.VMEM((_NCHUNK_A, _CH), jnp.int32),
        pltpu.VMEM((_CH, _DW), jnp.float32),
        pltpu.VMEM_SHARED((_N_PAD, _DW), jnp.float32),
        pltpu.SemaphoreType.DMA,
    ],
    compiler_params=_sc_params,
)
def _deg_kernel(dst_hbm, ones_hbm, zero_hbm, out_hbm, dst_v, ones_v, acc, dsem):
    c = lax.axis_index("c")
    s = lax.axis_index("s")
    pltpu.sync_copy(dst_hbm.at[s], dst_v)
    pltpu.sync_copy(ones_hbm, ones_v)
    my_rows = pl.ds(s * _RPS, _RPS)
    pltpu.sync_copy(zero_hbm.at[my_rows], acc.at[my_rows])
    plsc.subcore_barrier()

    half = _NCHUNK_A // _NC
    off = c * half

    def body(i, carry):
        j0 = off + i * 8
        for b in range(8):
            pltpu.async_copy(ones_v, acc.at[dst_v.at[j0 + b]], dsem, add=True)
        for b in range(8):
            pltpu.make_async_copy(ones_v, acc.at[dst_v.at[j0 + b]], dsem).wait()
        return carry

    lax.fori_loop(0, half // 8, body, 0)
    plsc.subcore_barrier()
    pltpu.sync_copy(acc.at[my_rows], out_hbm.at[c, my_rows])


@functools.partial(
    pl.kernel,
    out_type=jax.ShapeDtypeStruct((_NC, _N_PAD, _HH), jnp.float32),
    mesh=_mesh,
    scratch_types=[
        pltpu.VMEM((_EPS_A,), jnp.int32),
        pltpu.VMEM((_NCHUNK_A, _CH), jnp.int32),
        [pltpu.VMEM((_CH, _HH), jnp.float32)] * _NBUF,
        pltpu.VMEM_SHARED((_N_PAD, _HH), jnp.float32),
        pltpu.VMEM_SHARED((_N_PAD, _HH), jnp.float32),
        [pltpu.SemaphoreType.DMA] * _NBUF,
        [pltpu.SemaphoreType.DMA] * _NBUF,
    ],
    compiler_params=_sc_params,
)
def _agg_kernel(src_hbm, dst_hbm, hp_hbm, zero_hbm, out_hbm,
                src_v, dst_v, bufs, acc, hp_spm, gsems, ssems):
    c = lax.axis_index("c")
    s = lax.axis_index("s")

    pltpu.sync_copy(src_hbm.at[s], src_v)
    pltpu.sync_copy(dst_hbm.at[s], dst_v)

    my_rows = pl.ds(s * _RPS, _RPS)
    pltpu.sync_copy(hp_hbm.at[c, my_rows], hp_spm.at[my_rows])
    pltpu.sync_copy(zero_hbm.at[my_rows], acc.at[my_rows])
    plsc.subcore_barrier()

    def gather(j, b):
        pltpu.async_copy(
            hp_spm.at[src_v.at[pl.ds(j * _CH, _CH)]], bufs[b], gsems[b])

    def gwait(j, b):
        pltpu.make_async_copy(
            hp_spm.at[src_v.at[pl.ds(j * _CH, _CH)]], bufs[b], gsems[b]).wait()

    def scatter(j, b):
        pltpu.async_copy(bufs[b], acc.at[dst_v.at[j]], ssems[b], add=True)

    def swait(j, b):
        pltpu.make_async_copy(
            bufs[b], acc.at[dst_v.at[j]], ssems[b]).wait()

    for b in range(_LOOK):
        gather(b, b)

    def step(i, j, b, first):
        gwait(j, b)
        scatter(j, b)
        bn = (b + _LOOK) % _NBUF
        if first:
            if j >= _LOOK:
                swait(j - _LOOK, bn)
            gather(j + _LOOK, bn)
        else:
            swait(j - _LOOK, bn)

            @pl.when(j + _LOOK < _NCHUNK_A)
            def _():
                gather(j + _LOOK, bn)

    for b in range(_NBUF):
        step(0, b, b, True)

    def body(i, carry):
        j0 = i * _NBUF
        for b in range(_NBUF):
            step(i, j0 + b, b, False)
        return carry

    lax.fori_loop(1, _NCHUNK_A // _NBUF, body, 0)

    for b in range(_LOOK):
        swait(_NCHUNK_A - _LOOK + b, (_NCHUNK_A - _LOOK + b) % _NBUF)

    plsc.subcore_barrier()
    pltpu.sync_copy(acc.at[my_rows], out_hbm.at[c, my_rows])


_NB = 8
_BR = _N_PAD // _NB


def _hp1_body(parts_ref, x_ref, w_ref, dis_out, hp_out):
    deg = parts_ref[0, :, 0:1] + parts_ref[1, :, 0:1]
    dis = lax.rsqrt(deg + 1.0)
    dis_out[...] = dis
    h = jnp.dot(x_ref[...], w_ref[...], preferred_element_type=jnp.float32)
    hp = h * dis
    hp_out[0] = hp[:, :_HH]
    hp_out[1] = hp[:, _HH:]


def _hp1_call(deg_parts, x_pad, W1):
    f32 = jnp.float32
    return pl.pallas_call(
        _hp1_body,
        grid=(_NB,),
        in_specs=[
            pl.BlockSpec((2, _BR, _DW), lambda i: (0, i, 0)),
            pl.BlockSpec((_BR, _NODE_DIM), lambda i: (i, 0)),
            pl.BlockSpec((_NODE_DIM, _HID), lambda i: (0, 0)),
        ],
        out_specs=[
            pl.BlockSpec((_BR, 1), lambda i: (i, 0)),
            pl.BlockSpec((2, _BR, _HH), lambda i: (0, i, 0)),
        ],
        out_shape=[
            jax.ShapeDtypeStruct((_N_PAD, 1), f32),
            jax.ShapeDtypeStruct((_NC, _N_PAD, _HH), f32),
        ],
    )(deg_parts, x_pad, W1)


def _block_conv(parts_ref, hp_ref, dis_ref, b_ref):
    agg = jnp.concatenate([parts_ref[0], parts_ref[1]], axis=1)
    hpf = jnp.concatenate([hp_ref[0], hp_ref[1]], axis=1)
    return dis_ref[...] * (agg + hpf) + b_ref[...]


def _block_mask(conv, i):
    rowid = lax.broadcasted_iota(jnp.int32, (_BR, 1), 0) + i * _BR
    return jnp.where(rowid < _N, conv, 0.0)


def _bn_stats_step(conv, i, stats):
    s = jnp.stack([jnp.sum(conv, axis=0), jnp.sum(conv * conv, axis=0)])

    @pl.when(i == 0)
    def _():
        stats[...] = jnp.zeros_like(stats)

    stats[...] += s


def _bn_apply(conv, g_ref, be_ref, stats):
    mean = stats[0:1] * (1.0 / _N)
    var = stats[1:2] * (1.0 / _N) - mean * mean
    inv = lax.rsqrt(var + _EPS)
    a = g_ref[...] * (conv - mean) * inv + be_ref[...]
    return jnp.maximum(a, 0.0)


def _layer_body(parts_ref, hp_ref, dis_ref, b_ref, g_ref, be_ref, w_ref,
                out_ref, stats):
    p = pl.program_id(0)
    i = pl.program_id(1)
    conv = _block_mask(_block_conv(parts_ref, hp_ref, dis_ref, b_ref), i)

    @pl.when(p == 0)
    def _():
        _bn_stats_step(conv, i, stats)

    @pl.when(p == 1)
    def _():
        a = _block_mask(_bn_apply(conv, g_ref, be_ref, stats[...]), i)
        hp = jnp.dot(
            a, w_ref[...], preferred_element_type=jnp.float32) * dis_ref[...]
        out_ref[0] = hp[:, :_HH]
        out_ref[1] = hp[:, _HH:]


def _layer_call(parts, hp, dis_col, b, g, be, Wn):
    f32 = jnp.float32
    return pl.pallas_call(
        _layer_body,
        grid=(2, _NB),
        in_specs=[
            pl.BlockSpec((2, _BR, _HH), lambda p, i: (0, i, 0)),
            pl.BlockSpec((2, _BR, _HH), lambda p, i: (0, i, 0)),
            pl.BlockSpec((_BR, 1), lambda p, i: (i, 0)),
            pl.BlockSpec((1, _HID), lambda p, i: (0, 0)),
            pl.BlockSpec((1, _HID), lambda p, i: (0, 0)),
            pl.BlockSpec((1, _HID), lambda p, i: (0, 0)),
            pl.BlockSpec((_HID, _HID), lambda p, i: (0, 0)),
        ],
        out_specs=pl.BlockSpec((2, _BR, _HH), lambda p, i: (0, i, 0)),
        out_shape=jax.ShapeDtypeStruct((_NC, _N_PAD, _HH), f32),
        scratch_shapes=[pltpu.VMEM((2, _HID), f32)],
    )(parts, hp, dis_col, b, g, be, Wn)


def _final_body(parts_ref, hp_ref, dis_ref, b_ref, g_ref, be_ref, batch_ref,
                out_ref, stats, pool):
    p = pl.program_id(0)
    i = pl.program_id(1)
    conv = _block_mask(_block_conv(parts_ref, hp_ref, dis_ref, b_ref), i)

    @pl.when(p == 0)
    def _():
        _bn_stats_step(conv, i, stats)

    @pl.when(p == 1)
    def _():
        a = _block_mask(_bn_apply(conv, g_ref, be_ref, stats[...]), i)
        gid = lax.broadcasted_iota(jnp.int32, (1, _G), 1)
        onehot = (batch_ref[...] == gid).astype(jnp.float32)
        dnum = (((0,), (0,)), ((), ()))
        sums = lax.dot_general(onehot, a, dnum,
                               preferred_element_type=jnp.float32)
        ones_col = jnp.ones((_BR, 1), jnp.float32)
        cnt = lax.dot_general(onehot, ones_col, dnum,
                              preferred_element_type=jnp.float32)

        @pl.when(i == 0)
        def _():
            pool[...] = jnp.zeros_like(pool)

        pool[...] += jnp.concatenate([sums, cnt], axis=1)

        @pl.when(i == _NB - 1)
        def _():
            out_ref[...] = pool[:, :_HID] / jnp.maximum(pool[:, _HID:], 1.0)


def _final_call(parts, hp, dis_col, b, g, be, batch_col):
    f32 = jnp.float32
    return pl.pallas_call(
        _final_body,
        grid=(2, _NB),
        in_specs=[
            pl.BlockSpec((2, _BR, _HH), lambda p, i: (0, i, 0)),
            pl.BlockSpec((2, _BR, _HH), lambda p, i: (0, i, 0)),
            pl.BlockSpec((_BR, 1), lambda p, i: (i, 0)),
            pl.BlockSpec((1, _HID), lambda p, i: (0, 0)),
            pl.BlockSpec((1, _HID), lambda p, i: (0, 0)),
            pl.BlockSpec((1, _HID), lambda p, i: (0, 0)),
            pl.BlockSpec((_BR, 1), lambda p, i: (i, 0)),
        ],
        out_specs=pl.BlockSpec((_G, _HID), lambda p, i: (0, 0)),
        out_shape=jax.ShapeDtypeStruct((_G, _HID), f32),
        scratch_shapes=[
            pltpu.VMEM((2, _HID), f32),
            pltpu.VMEM((_G, _HID + 1), f32),
        ],
    )(parts, hp, dis_col, b, g, be, batch_col)


def kernel(x, edge_index, batch, W1, b1, g1, be1, W2, b2, g2, be2, W3, b3, g3, be3):
    f32 = jnp.float32
    x_pad = jnp.pad(x, ((0, _N_PAD - _N), (0, 0)))
    epad = jnp.full((_E_PAD - _E,), _N, jnp.int32)
    src_flat = jnp.concatenate([edge_index[0], epad]).reshape(_NS, _EPS_A)
    dst_tiled = jnp.concatenate(
        [edge_index[1], epad]).reshape(_NS, _NCHUNK_A, _CH)
    batch_col = jnp.concatenate(
        [batch, jnp.full((_N_PAD - _N,), _G, jnp.int32)]).reshape(_N_PAD, 1)
    zeros2d = jnp.zeros((_N_PAD, _HH), f32)
    ones8 = jnp.ones((_CH, _DW), f32)
    zeros8 = jnp.zeros((_N_PAD, _DW), f32)

    deg_parts = _deg_kernel(dst_tiled, ones8, zeros8)
    dis_col, hp = _hp1_call(deg_parts, x_pad, W1)

    for (b, g, be, Wn) in ((b1, g1, be1, W2), (b2, g2, be2, W3)):
        parts = _agg_kernel(src_flat, dst_tiled, hp, zeros2d)
        hp = _layer_call(parts, hp, dis_col, b.reshape(1, _HID),
                         g.reshape(1, _HID), be.reshape(1, _HID), Wn)

    parts = _agg_kernel(src_flat, dst_tiled, hp, zeros2d)
    return _final_call(parts, hp, dis_col, b3.reshape(1, _HID),
                       g3.reshape(1, _HID), be3.reshape(1, _HID), batch_col)


# speedup vs baseline: 1.1767x; 1.1767x over previous
import functools

import jax
import jax.numpy as jnp
from jax import lax
from jax.experimental import pallas as pl
from jax.experimental.pallas import tpu as pltpu
from jax.experimental.pallas import tpu_sc as plsc

_N = 10000
_E = 320000
_NODE_DIM = 128
_HID = 64
_G = 64
_EPS = 1e-5

_NC = 2
_NS = 16
_NW = _NC * _NS
_CH = 128
_NCHUNK = 80
_EPT = _NCHUNK * _CH
_E_PAD = _NW * _EPT
_N_PAD = 10240
_RPS = _N_PAD // _NS
_NBUF = 8
_LOOK = 4
_HH = _HID // 2
_NCHUNK_A = 160
_EPS_A = _NCHUNK_A * _CH

_mesh = plsc.VectorSubcoreMesh(
    core_axis_name="c", subcore_axis_name="s", num_cores=_NC, num_subcores=_NS)
_sc_params = pltpu.CompilerParams(use_tc_tiling_on_sc=False)


_DW = 8


@functools.partial(
    pl.kernel,
    out_type=jax.ShapeDtypeStruct((_NC, _N_PAD, _DW), jnp.float32),
    mesh=_mesh,
    scratch_types=[
        pltpu.VMEM((_NCHUNK_A, _CH), jnp.int32),
        pltpu.VMEM((_CH, _DW), jnp.float32),
        pltpu.VMEM_SHARED((_N_PAD, _DW), jnp.float32),
        pltpu.SemaphoreType.DMA,
    ],
    compiler_params=_sc_params,
)
def _deg_kernel(dst_hbm, ones_hbm, zero_hbm, out_hbm, dst_v, ones_v, acc, dsem):
    c = lax.axis_index("c")
    s = lax.axis_index("s")
    pltpu.sync_copy(dst_hbm.at[s], dst_v)
    pltpu.sync_copy(ones_hbm, ones_v)
    my_rows = pl.ds(s * _RPS, _RPS)
    pltpu.sync_copy(zero_hbm.at[my_rows], acc.at[my_rows])
    plsc.subcore_barrier()

    half = _NCHUNK_A // _NC
    off = c * half

    def body(i, carry):
        j0 = off + i * 8
        for b in range(8):
            pltpu.async_copy(ones_v, acc.at[dst_v.at[j0 + b]], dsem, add=True)
        for b in range(8):
            pltpu.make_async_copy(ones_v, acc.at[dst_v.at[j0 + b]], dsem).wait()
        return carry

    lax.fori_loop(0, half // 8, body, 0)
    plsc.subcore_barrier()
    pltpu.sync_copy(acc.at[my_rows], out_hbm.at[c, my_rows])


@functools.partial(
    pl.kernel,
    out_type=jax.ShapeDtypeStruct((_N_PAD, _HID), jnp.float32),
    mesh=_mesh,
    scratch_types=[
        pltpu.VMEM((_EPS_A,), jnp.int32),
        pltpu.VMEM((_NCHUNK_A, _CH), jnp.int32),
        [pltpu.VMEM((_CH, _HH), jnp.float32)] * _NBUF,
        pltpu.VMEM_SHARED((_N_PAD, _HH), jnp.float32),
        pltpu.VMEM_SHARED((_N_PAD, _HH), jnp.float32),
        [pltpu.SemaphoreType.DMA] * _NBUF,
        [pltpu.SemaphoreType.DMA] * _NBUF,
    ],
    compiler_params=_sc_params,
)
def _agg_kernel(src_hbm, dst_hbm, hp_hbm, zero_hbm, out_hbm,
                src_v, dst_v, bufs, acc, hp_spm, gsems, ssems):
    c = lax.axis_index("c")
    s = lax.axis_index("s")

    pltpu.sync_copy(src_hbm.at[s], src_v)
    pltpu.sync_copy(dst_hbm.at[s], dst_v)

    my_rows = pl.ds(s * _RPS, _RPS)
    my_cols = pl.ds(c * _HH, _HH)
    pltpu.sync_copy(hp_hbm.at[my_rows, my_cols], hp_spm.at[my_rows])
    pltpu.sync_copy(zero_hbm.at[my_rows], acc.at[my_rows])
    plsc.subcore_barrier()

    def gather(j, b):
        pltpu.async_copy(
            hp_spm.at[src_v.at[pl.ds(j * _CH, _CH)]], bufs[b], gsems[b])

    def gwait(j, b):
        pltpu.make_async_copy(
            hp_spm.at[src_v.at[pl.ds(j * _CH, _CH)]], bufs[b], gsems[b]).wait()

    def scatter(j, b):
        pltpu.async_copy(bufs[b], acc.at[dst_v.at[j]], ssems[b], add=True)

    def swait(j, b):
        pltpu.make_async_copy(
            bufs[b], acc.at[dst_v.at[j]], ssems[b]).wait()

    for b in range(_LOOK):
        gather(b, b)

    def step(i, j, b, first):
        gwait(j, b)
        scatter(j, b)
        bn = (b + _LOOK) % _NBUF
        if first:
            if j >= _LOOK:
                swait(j - _LOOK, bn)
            gather(j + _LOOK, bn)
        else:
            swait(j - _LOOK, bn)

            @pl.when(j + _LOOK < _NCHUNK_A)
            def _():
                gather(j + _LOOK, bn)

    for b in range(_NBUF):
        step(0, b, b, True)

    def body(i, carry):
        j0 = i * _NBUF
        for b in range(_NBUF):
            step(i, j0 + b, b, False)
        return carry

    lax.fori_loop(1, _NCHUNK_A // _NBUF, body, 0)

    for b in range(_LOOK):
        swait(_NCHUNK_A - _LOOK + b, (_NCHUNK_A - _LOOK + b) % _NBUF)

    plsc.subcore_barrier()
    pltpu.sync_copy(acc.at[my_rows], out_hbm.at[my_rows, my_cols])


_NB = 8
_BR = _N_PAD // _NB


def _hp1_body(parts_ref, x_ref, w_ref, dis_out, hp_out):
    deg = parts_ref[0, :, 0:1] + parts_ref[1, :, 0:1]
    dis = lax.rsqrt(deg + 1.0)
    dis_out[...] = dis
    h = jnp.dot(x_ref[...], w_ref[...], preferred_element_type=jnp.float32)
    hp_out[...] = h * dis


def _hp1_call(deg_parts, x_pad, W1):
    f32 = jnp.float32
    return pl.pallas_call(
        _hp1_body,
        grid=(_NB,),
        in_specs=[
            pl.BlockSpec((2, _BR, _DW), lambda i: (0, i, 0)),
            pl.BlockSpec((_BR, _NODE_DIM), lambda i: (i, 0)),
            pl.BlockSpec((_NODE_DIM, _HID), lambda i: (0, 0)),
        ],
        out_specs=[
            pl.BlockSpec((_BR, 1), lambda i: (i, 0)),
            pl.BlockSpec((_BR, _HID), lambda i: (i, 0)),
        ],
        out_shape=[
            jax.ShapeDtypeStruct((_N_PAD, 1), f32),
            jax.ShapeDtypeStruct((_N_PAD, _HID), f32),
        ],
    )(deg_parts, x_pad, W1)


def _bn_relu(conv, g_ref, be_ref):
    cr = conv[:_N]
    mean = jnp.sum(cr, axis=0) * (1.0 / _N)
    var = jnp.sum((cr - mean) ** 2, axis=0) * (1.0 / _N)
    inv = lax.rsqrt(var + _EPS)
    a = g_ref[...] * (conv - mean) * inv + be_ref[...]
    a = jnp.maximum(a, 0.0)
    rowid = lax.broadcasted_iota(jnp.int32, (_N_PAD, 1), 0)
    return jnp.where(rowid < _N, a, 0.0)


def _layer_body(parts_ref, hp_ref, dis_ref, b_ref, g_ref, be_ref, w_ref, out_ref):
    conv = dis_ref[...] * (parts_ref[...] + hp_ref[...]) + b_ref[...]
    a = _bn_relu(conv, g_ref, be_ref)
    out_ref[...] = jnp.dot(
        a, w_ref[...], preferred_element_type=jnp.float32) * dis_ref[...]


def _layer_call(parts, hp, dis_col, b, g, be, Wn):
    f32 = jnp.float32
    return pl.pallas_call(
        _layer_body,
        out_shape=jax.ShapeDtypeStruct((_N_PAD, _HID), f32),
    )(parts, hp, dis_col, b, g, be, Wn)


def _final_body(parts_ref, hp_ref, dis_ref, b_ref, g_ref, be_ref, batch_ref,
                out_ref):
    conv = dis_ref[...] * (parts_ref[...] + hp_ref[...]) + b_ref[...]
    a = _bn_relu(conv, g_ref, be_ref)
    gid = lax.broadcasted_iota(jnp.int32, (1, _G), 1)
    onehot = (batch_ref[...] == gid).astype(jnp.float32)
    dnum = (((0,), (0,)), ((), ()))
    sums = lax.dot_general(onehot, a, dnum, preferred_element_type=jnp.float32)
    ones_col = jnp.ones((_N_PAD, 1), jnp.float32)
    cnt = lax.dot_general(onehot, ones_col, dnum,
                          preferred_element_type=jnp.float32)
    out_ref[...] = sums / jnp.maximum(cnt, 1.0)


def _final_call(parts, hp, dis_col, b, g, be, batch_col):
    f32 = jnp.float32
    return pl.pallas_call(
        _final_body,
        out_shape=jax.ShapeDtypeStruct((_G, _HID), f32),
    )(parts, hp, dis_col, b, g, be, batch_col)


def kernel(x, edge_index, batch, W1, b1, g1, be1, W2, b2, g2, be2, W3, b3, g3, be3):
    f32 = jnp.float32
    x_pad = jnp.pad(x, ((0, _N_PAD - _N), (0, 0)))
    epad = jnp.full((_E_PAD - _E,), _N, jnp.int32)
    src_flat = jnp.concatenate([edge_index[0], epad]).reshape(_NS, _EPS_A)
    dst_tiled = jnp.concatenate(
        [edge_index[1], epad]).reshape(_NS, _NCHUNK_A, _CH)
    batch_col = jnp.concatenate(
        [batch, jnp.full((_N_PAD - _N,), _G, jnp.int32)]).reshape(_N_PAD, 1)
    zeros2d = jnp.zeros((_N_PAD, _HH), f32)
    ones8 = jnp.ones((_CH, _DW), f32)
    zeros8 = jnp.zeros((_N_PAD, _DW), f32)

    deg_parts = _deg_kernel(dst_tiled, ones8, zeros8)
    dis_col, hp = _hp1_call(deg_parts, x_pad, W1)

    for (b, g, be, Wn) in ((b1, g1, be1, W2), (b2, g2, be2, W3)):
        parts = _agg_kernel(src_flat, dst_tiled, hp, zeros2d)
        hp = _layer_call(parts, hp, dis_col, b.reshape(1, _HID),
                         g.reshape(1, _HID), be.reshape(1, _HID), Wn)

    parts = _agg_kernel(src_flat, dst_tiled, hp, zeros2d)
    return _final_call(parts, hp, dis_col, b3.reshape(1, _HID),
                       g3.reshape(1, _HID), be3.reshape(1, _HID), batch_col)
